# baseline (speedup 1.0000x reference)
import functools

import jax
import jax.numpy as jnp
import numpy as np
from jax import lax
from jax.experimental import pallas as pl
from jax.experimental.pallas import tpu as pltpu
from jax.experimental.pallas import tpu_sc as plsc

N = 10000
E = 320000
NODE_DIM = 128
PE_DIM = 10
BOND_DIM = 16
HID = 128
N_BINS = 32
N_SITES = 10
PKA_MEAN = 7.0
PKA_STD = 3.0

_ROWS = 400
_RSQRT_H = float(1.0 / np.sqrt(float(HID)))

NW = 32
WIN = 48
DEPTH = 6
NPASS = 6
NCUT = NW * NPASS
STG = 16
RANGE_P = 64
E_PAD = E + WIN
_NCHUNK = HID // 16


def _t_cut(j):
    return 16 * ((j * (N // 16)) // NCUT)


def _h0_body(x_ref, lap_ref, wx_ref, wpe_ref, o_ref):
    o_ref[...] = (
        jnp.dot(x_ref[...], wx_ref[...], preferred_element_type=jnp.float32)
        + jnp.dot(lap_ref[...], wpe_ref[...], preferred_element_type=jnp.float32)
    )


def _h0_matmul(x, lap_pad, wx, wpe_pad):
    return pl.pallas_call(
        _h0_body,
        grid=(N // _ROWS,),
        in_specs=[
            pl.BlockSpec((_ROWS, NODE_DIM), lambda i: (i, 0)),
            pl.BlockSpec((_ROWS, 16), lambda i: (i, 0)),
            pl.BlockSpec((NODE_DIM, HID), lambda i: (0, 0)),
            pl.BlockSpec((16, HID), lambda i: (0, 0)),
        ],
        out_specs=pl.BlockSpec((_ROWS, HID), lambda i: (i, 0)),
        out_shape=jax.ShapeDtypeStruct((N, HID), jnp.float32),
    )(x, lap_pad, wx, wpe_pad)


def _qkvsu_body(h_ref, wq_ref, bq_ref, wkv_ref, bkv_ref, ws_ref, bs_ref,
                wet_ref, qu_ref, kv_ref, s_ref):
    h = h_ref[...]
    q = jnp.dot(h, wq_ref[...], preferred_element_type=jnp.float32) + bq_ref[...]
    u = jnp.dot(q, wet_ref[...], preferred_element_type=jnp.float32)
    qu_ref[...] = jnp.concatenate(
        [q, u, jnp.zeros((q.shape[0], 2 * HID - HID - BOND_DIM), jnp.float32)], axis=1)
    kv_ref[...] = jnp.dot(h, wkv_ref[...], preferred_element_type=jnp.float32) + bkv_ref[...]
    s_ref[...] = jnp.dot(h, ws_ref[...], preferred_element_type=jnp.float32) + bs_ref[...]


def _qkvsu_matmul(h, wq, bq, wkv, bkv, ws, bs, wet):
    return pl.pallas_call(
        _qkvsu_body,
        grid=(N // _ROWS,),
        in_specs=[
            pl.BlockSpec((_ROWS, HID), lambda i: (i, 0)),
            pl.BlockSpec((HID, HID), lambda i: (0, 0)),
            pl.BlockSpec((1, HID), lambda i: (0, 0)),
            pl.BlockSpec((HID, 2 * HID), lambda i: (0, 0)),
            pl.BlockSpec((1, 2 * HID), lambda i: (0, 0)),
            pl.BlockSpec((HID, HID), lambda i: (0, 0)),
            pl.BlockSpec((1, HID), lambda i: (0, 0)),
            pl.BlockSpec((HID, BOND_DIM), lambda i: (0, 0)),
        ],
        out_specs=[
            pl.BlockSpec((_ROWS, 2 * HID), lambda i: (i, 0)),
            pl.BlockSpec((_ROWS, 2 * HID), lambda i: (i, 0)),
            pl.BlockSpec((_ROWS, HID), lambda i: (i, 0)),
        ],
        out_shape=[
            jax.ShapeDtypeStruct((N, 2 * HID), jnp.float32),
            jax.ShapeDtypeStruct((N, 2 * HID), jnp.float32),
            jax.ShapeDtypeStruct((N, HID), jnp.float32),
        ],
    )(h, wq, bq, wkv, bkv, ws, bs, wet)


def _heads_body(h_ref, wc1_ref, bc1_ref, wc2_ref, bc2_ref,
                wr1_ref, br1_ref, wr2_ref, br2_ref, wg_ref, bg_ref,
                lg_ref, pr_ref, gt_ref):
    h = h_ref[...]
    z = jax.nn.relu(jnp.dot(h, wc1_ref[...], preferred_element_type=jnp.float32) + bc1_ref[...])
    lg_ref[...] = jnp.dot(z, wc2_ref[...], preferred_element_type=jnp.float32) + bc2_ref[...]
    r = jax.nn.relu(jnp.dot(h, wr1_ref[...], preferred_element_type=jnp.float32) + br1_ref[...])
    pr_ref[...] = jnp.dot(r, wr2_ref[...], preferred_element_type=jnp.float32) + br2_ref[...]
    gt_ref[...] = jnp.tanh(jnp.dot(h, wg_ref[...], preferred_element_type=jnp.float32) + bg_ref[...])


def _heads_matmul(h, wc1, bc1, wc2p, bc2p, wr1, br1, wr2p, br2p, wg, bg):
    return pl.pallas_call(
        _heads_body,
        grid=(N // _ROWS,),
        in_specs=[
            pl.BlockSpec((_ROWS, HID), lambda i: (i, 0)),
            pl.BlockSpec((HID, 128), lambda i: (0, 0)),
            pl.BlockSpec((1, 128), lambda i: (0, 0)),
            pl.BlockSpec((128, 128), lambda i: (0, 0)),
            pl.BlockSpec((1, 128), lambda i: (0, 0)),
            pl.BlockSpec((HID, 128), lambda i: (0, 0)),
            pl.BlockSpec((1, 128), lambda i: (0, 0)),
            pl.BlockSpec((128, 128), lambda i: (0, 0)),
            pl.BlockSpec((1, 128), lambda i: (0, 0)),
            pl.BlockSpec((HID, HID), lambda i: (0, 0)),
            pl.BlockSpec((1, HID), lambda i: (0, 0)),
        ],
        out_specs=[
            pl.BlockSpec((_ROWS, 128), lambda i: (i, 0)),
            pl.BlockSpec((_ROWS, 128), lambda i: (i, 0)),
            pl.BlockSpec((_ROWS, HID), lambda i: (i, 0)),
        ],
        out_shape=[
            jax.ShapeDtypeStruct((N, 128), jnp.float32),
            jax.ShapeDtypeStruct((N, 128), jnp.float32),
            jax.ShapeDtypeStruct((N, HID), jnp.float32),
        ],
    )(h, wc1, bc1, wc2p, bc2p, wr1, br1, wr2p, br2p, wg, bg)


def _combine_body(s1_ref, s2_ref, we_ref, skip_ref, o_ref):
    o_ref[...] = (
        s1_ref[...]
        + jnp.dot(s2_ref[...][:, :BOND_DIM], we_ref[...],
                  preferred_element_type=jnp.float32)
        + skip_ref[...]
    )


def _combine_matmul(s1, s2, we, skip):
    return pl.pallas_call(
        _combine_body,
        grid=(N // _ROWS,),
        in_specs=[
            pl.BlockSpec((_ROWS, HID), lambda i: (i, 0)),
            pl.BlockSpec((_ROWS, 128), lambda i: (i, 0)),
            pl.BlockSpec((BOND_DIM, HID), lambda i: (0, 0)),
            pl.BlockSpec((_ROWS, HID), lambda i: (i, 0)),
        ],
        out_specs=pl.BlockSpec((_ROWS, HID), lambda i: (i, 0)),
        out_shape=jax.ShapeDtypeStruct((N, HID), jnp.float32),
    )(s1, s2, we, skip)


_MESH = plsc.VectorSubcoreMesh(core_axis_name="c", subcore_axis_name="s")


@functools.partial(
    pl.kernel,
    out_type=[
        jax.ShapeDtypeStruct((N, HID), jnp.float32),
        jax.ShapeDtypeStruct((N, 128), jnp.float32),
    ],
    mesh=_MESH,
    scratch_types=[
        pltpu.VMEM((RANGE_P, 2 * HID), jnp.float32),
        pltpu.VMEM((RANGE_P, HID), jnp.float32),
        pltpu.VMEM((RANGE_P, 128), jnp.float32),
        pltpu.VMEM((DEPTH, WIN), jnp.int32),
        pltpu.VMEM((DEPTH, WIN), jnp.int32),
        pltpu.VMEM((DEPTH * WIN * BOND_DIM,), jnp.float32),
        pltpu.VMEM((DEPTH, WIN, 2 * HID), jnp.float32),
        pltpu.VMEM((224,), jnp.int32),
        pltpu.SemaphoreType.DMA,
        pltpu.SemaphoreType.DMA,
        pltpu.SemaphoreType.DMA,
        pltpu.SemaphoreType.DMA,
        pltpu.SemaphoreType.DMA,
        pltpu.SemaphoreType.DMA,
        pltpu.SemaphoreType.DMA,
        pltpu.SemaphoreType.DMA,
        pltpu.SemaphoreType.DMA,
        pltpu.SemaphoreType.DMA,
        pltpu.SemaphoreType.DMA,
        pltpu.SemaphoreType.DMA,
    ],
)
def _edge_sc(qu_hbm, src_hbm, dst_hbm, ea_hbm, kv_hbm, starts_hbm,
             s1_hbm, s2_hbm,
             qub, t1b, auxb, srcw, dstw, eaw, kvw, stbl,
             sem_src0, sem_src1, sem_src2, sem_src3, sem_src4, sem_src5,
             sem_kv0, sem_kv1, sem_kv2, sem_kv3, sem_kv4, sem_kv5):
    wid = lax.axis_index("s") * 2 + lax.axis_index("c")

    pltpu.sync_copy(starts_hbm, stbl)

    sem_src = (sem_src0, sem_src1, sem_src2, sem_src3, sem_src4, sem_src5)
    sem_kv = (sem_kv0, sem_kv1, sem_kv2, sem_kv3, sem_kv4, sem_kv5)
    zrow = jnp.zeros((16,), jnp.float32)
    mrow = jnp.full((16,), -3e38, jnp.float32)

    def run_pass(pi, carry):
        j = wid * NPASS + pi
        t_lo = 16 * ((j * (N // 16)) // NCUT)
        t_hi = 16 * (((j + 1) * (N // 16)) // NCUT)
        sv = stbl[pl.ds(j, 16)]
        a_w = (sv[0] // 8) * 8
        end_w = sv[1]
        nw = (end_w - a_w + (WIN - 1)) // WIN

        pltpu.sync_copy(qu_hbm.at[pl.ds(t_lo, RANGE_P)], qub)

        def init_body(r, c2):
            for c in range(_NCHUNK):
                t1b[r, pl.ds(16 * c, 16)] = zrow
            auxb[r, pl.ds(0, 16)] = zrow
            auxb[r, pl.ds(16, 16)] = mrow
            auxb[r, pl.ds(32, 16)] = zrow
            return c2

        lax.fori_loop(0, RANGE_P, init_body, 0)

        def issue_src(g, par):
            e0 = a_w + g * WIN
            pltpu.make_async_copy(src_hbm.at[pl.ds(e0, WIN)], srcw.at[par],
                                  sem_src[par]).start()
            pltpu.make_async_copy(dst_hbm.at[pl.ds(e0, WIN)], dstw.at[par],
                                  sem_src[par]).start()
            pltpu.make_async_copy(ea_hbm.at[pl.ds(e0 * BOND_DIM, WIN * BOND_DIM)],
                                  eaw.at[pl.ds(par * WIN * BOND_DIM, WIN * BOND_DIM)],
                                  sem_src[par]).start()

        def wait_src(par):
            pltpu.make_async_copy(src_hbm.at[pl.ds(0, WIN)], srcw.at[par],
                                  sem_src[par]).wait()
            pltpu.make_async_copy(dst_hbm.at[pl.ds(0, WIN)], dstw.at[par],
                                  sem_src[par]).wait()
            pltpu.make_async_copy(ea_hbm.at[pl.ds(0, WIN * BOND_DIM)],
                                  eaw.at[pl.ds(par * WIN * BOND_DIM, WIN * BOND_DIM)],
                                  sem_src[par]).wait()

        def issue_kv(par):
            pltpu.make_async_copy(kv_hbm.at[srcw.at[par]], kvw.at[par],
                                  sem_kv[par]).start()

        def wait_kv(par):
            pltpu.make_async_copy(kv_hbm.at[srcw.at[par]], kvw.at[par],
                                  sem_kv[par]).wait()

        def win_body(g, par):
            wait_kv(par)

            def group_body(gi, c2):
                base = gi * 16
                dvec = dstw[par, pl.ds(base, 16)]
                lane = lax.iota(jnp.int32, 16)

                alpha16 = zrow
                d_eff = []
                for l in range(16):
                    i = base + l
                    d = dvec[l]
                    valid = jnp.logical_and(d >= t_lo, d < t_hi)
                    row = jnp.where(valid, d - t_lo, 0)
                    d_eff.append(jnp.minimum(jnp.maximum(d, t_lo), t_hi - 1))
                    ea_row = eaw[pl.ds(par * WIN * BOND_DIM + i * BOND_DIM, 16)]
                    acc = qub[row, pl.ds(HID, BOND_DIM)] * ea_row
                    for c in range(_NCHUNK):
                        acc = acc + qub[row, pl.ds(16 * c, 16)] * kvw[par, i, pl.ds(16 * c, 16)]
                    for k in (8, 4, 2, 1):
                        acc = acc + acc.at[lane ^ k].get(mode="promise_in_bounds")
                    vf = jnp.full((16,), jnp.where(valid, 1.0, 0.0), jnp.float32)
                    a_spl = acc * _RSQRT_H * vf + mrow * (1.0 - vf)
                    oh = jnp.maximum(1.0 - jnp.abs(lane - l).astype(jnp.float32), 0.0)
                    alpha16 = alpha16 + a_spl * oh

                mg = alpha16
                for k in (8, 4, 2, 1):
                    mg = jnp.maximum(mg, mg.at[lane ^ k].get(mode="promise_in_bounds"))
                gate = jnp.full(
                    (16,), jnp.where(mg[0] > -1e30, 1.0, 0.0), jnp.float32)
                exg = jnp.exp(alpha16 - mg) * gate

                def flush(prow, st):
                    accD, accT2 = st[0], st[1]
                    accT1 = st[2:]
                    row = prow - t_lo
                    m_old = auxb[row, pl.ds(16, 16)]
                    m_new = jnp.maximum(m_old, mg)
                    f_old = jnp.exp(m_old - m_new)
                    f_g = jnp.exp(mg - m_new)
                    auxb[row, pl.ds(16, 16)] = m_new
                    auxb[row, pl.ds(32, 16)] = auxb[row, pl.ds(32, 16)] * f_old + accD * f_g
                    auxb[row, pl.ds(0, 16)] = auxb[row, pl.ds(0, 16)] * f_old + accT2 * f_g
                    for c in range(_NCHUNK):
                        t1b[row, pl.ds(16 * c, 16)] = (
                            t1b[row, pl.ds(16 * c, 16)] * f_old + accT1[c] * f_g
                        )
                    return (zrow,) * (2 + _NCHUNK)

                st = (zrow,) * (2 + _NCHUNK)
                for l in range(16):
                    i = base + l
                    if l > 0:
                        st = lax.cond(d_eff[l] != d_eff[l - 1],
                                      lambda s, _r=d_eff[l - 1]: flush(_r, s),
                                      lambda s: s, st)
                    ex_s = jnp.full((16,), exg[l], jnp.float32)
                    ea_row = eaw[pl.ds(par * WIN * BOND_DIM + i * BOND_DIM, 16)]
                    new0 = st[0] + ex_s
                    new1 = st[1] + ex_s * ea_row
                    newT = tuple(
                        st[2 + c] + ex_s * kvw[par, i, pl.ds(HID + 16 * c, 16)]
                        for c in range(_NCHUNK))
                    st = (new0, new1) + newT
                flush(d_eff[15], st)
                return c2

            lax.fori_loop(0, WIN // 16, group_body, 0)

            @pl.when(g + DEPTH < nw)
            def _():
                issue_src(g + DEPTH, par)

            @pl.when(g + (DEPTH - 1) < nw)
            def _():
                wait_src((par + DEPTH - 1) % DEPTH)
                issue_kv((par + DEPTH - 1) % DEPTH)

        for i0 in range(DEPTH):
            @pl.when(i0 < nw)
            def _():
                issue_src(i0, i0)

        for i0 in range(DEPTH - 1):
            @pl.when(i0 < nw)
            def _():
                wait_src(i0)
                issue_kv(i0)

        def quad_body(gq, c2):
            g = DEPTH * gq
            for p in range(DEPTH):
                @pl.when(g + p < nw)
                def _():
                    win_body(g + p, p)
            return c2

        lax.fori_loop(0, (nw + DEPTH - 1) // DEPTH, quad_body, 0)

        def div_body(r, c2):
            inv = 1.0 / (auxb[r, pl.ds(32, 16)] + 1e-16)
            for c in range(_NCHUNK):
                t1b[r, pl.ds(16 * c, 16)] = t1b[r, pl.ds(16 * c, 16)] * inv
            auxb[r, pl.ds(0, 16)] = auxb[r, pl.ds(0, 16)] * inv
            return c2

        lax.fori_loop(0, t_hi - t_lo, div_body, 0)

        def flush_body(fb, c2):
            rb = fb * STG
            pltpu.sync_copy(t1b.at[pl.ds(rb, STG)], s1_hbm.at[pl.ds(t_lo + rb, STG)])
            pltpu.sync_copy(auxb.at[pl.ds(rb, STG)], s2_hbm.at[pl.ds(t_lo + rb, STG)])
            return c2

        lax.fori_loop(0, (t_hi - t_lo) // STG, flush_body, 0)
        return carry

    lax.fori_loop(0, NPASS, run_pass, 0)


def _prep(pka_labels):
    pos = pka_labels > 0
    sort_key = jnp.where(pos, pka_labels, jnp.inf)
    idx_sorted = jnp.argsort(sort_key, stable=True)[:N_SITES].astype(jnp.int32)
    target0 = pos.astype(jnp.int32)

    def step(cur, i):
        nxt = cur.at[i].set(0)
        return nxt, nxt

    _, rows = jax.lax.scan(step, target0, idx_sorted)
    target_final = jnp.concatenate([target0[None], rows], axis=0)
    return idx_sorted, target_final


def kernel(x, lap_pos, edge_attr, params, edge_index, pka_labels):
    p = params
    idx_sorted, target_final = _prep(pka_labels)
    steps = N_SITES + 1

    wkv = jnp.concatenate([p['Wk'], p['Wv']], axis=1)
    bkv = jnp.concatenate([p['bk'], p['bv']])[None, :]
    wc2p = jnp.zeros((128, 128), jnp.float32).at[:, :2].set(p['Wc2'])
    bc2p = jnp.zeros((1, 128), jnp.float32).at[0, :2].set(p['bc2'])
    wr2p = jnp.zeros((128, 128), jnp.float32).at[:, :1].set(p['Wr2'])
    br2p = jnp.zeros((1, 128), jnp.float32).at[0, :1].set(p['br2'])
    lap_pad = jnp.zeros((N, 16), jnp.float32).at[:, :PE_DIM].set(lap_pos)
    wpe_pad = jnp.zeros((16, HID), jnp.float32).at[:PE_DIM, :].set(p['W_pe'])
    wet = p['We'].T

    src0 = edge_index[0]
    dst0 = edge_index[1]
    order = jnp.argsort(dst0, stable=True)
    src = jnp.concatenate([src0[order], jnp.zeros((WIN,), jnp.int32)])
    dst = jnp.concatenate([dst0[order], jnp.full((WIN,), N, jnp.int32)])
    ea_s = jnp.concatenate(
        [edge_attr[order], jnp.zeros((WIN, BOND_DIM), jnp.float32)]).reshape(-1)
    t_cuts = jnp.asarray([_t_cut(j) for j in range(NCUT + 1)], jnp.int32)
    starts = jnp.searchsorted(dst[:E], t_cuts).astype(jnp.int32)
    starts = jnp.concatenate([starts, jnp.zeros((224 - (NCUT + 1),), jnp.int32)])

    h0 = _h0_matmul(x, lap_pad, p['W_x'], wpe_pad)

    def conv(h_in):
        qu, kv, skip = _qkvsu_matmul(
            h_in, p['Wq'], p['bq'][None, :], wkv, bkv,
            p['Wskip'], p['bskip'][None, :], wet)
        s1, s2 = _edge_sc(qu, src, dst, ea_s, kv, starts)
        return _combine_matmul(s1, s2, p['We'], skip)

    h = conv(h0)

    loss_cla_steps = []
    loss_reg_steps = []
    logitss = []
    pkas = []
    for si in range(steps):
        idx = idx_sorted[si] if si != steps - 1 else jnp.int32(-1)
        lg_p, pr_p, gate_all = _heads_matmul(
            h, p['Wc1'], p['bc1'][None, :], wc2p, bc2p,
            p['Wr1'], p['br1'][None, :], wr2p, br2p, p['Wg'], p['bg'][None, :])
        logits = lg_p[:, :2]
        logitss.append(logits)
        tgt = target_final[si]
        n1 = jnp.sum(tgt).astype(jnp.float32)
        n0 = jnp.sum(tgt == 0).astype(jnp.float32)
        ratio = n0 / (n1 + 1e-06)
        logp = jax.nn.log_softmax(logits, axis=-1)
        nll = -jnp.take_along_axis(logp, tgt[:, None], axis=1)[:, 0]
        wpn = jnp.where(tgt == 1, ratio, 1.0)
        loss_cla_steps.append(wpn * nll)
        if si != steps - 1:
            pred_pka = pr_p[idx, 0]
            true_pka = pka_labels[idx]
            bi = jnp.searchsorted(jax.lax.stop_gradient(p['bin_edges']),
                                  jax.lax.stop_gradient(true_pka), side='left') - 1
            w = p['bin_weights'][bi]
            loss_r = w * (((pred_pka - PKA_MEAN) / PKA_STD - (true_pka - PKA_MEAN) / PKA_STD) ** 2)
            loss_reg_steps.append(loss_r)
            pkas.append(pred_pka)
        gate = gate_all[idx]
        h_upd = h.at[idx].set(h0[idx] + h[idx] * gate)
        if si != steps - 1:
            h = conv(h_upd)
    loss_cla = jnp.mean(jnp.stack(loss_cla_steps))
    loss_reg = jnp.mean(jnp.stack(loss_reg_steps))
    total = loss_cla + loss_reg
    return total, loss_cla, loss_reg, jnp.stack(logitss), jnp.stack(pkas), target_final


# speedup vs baseline: 1.1940x; 1.1940x over previous
import functools

import jax
import jax.numpy as jnp
import numpy as np
from jax import lax
from jax.experimental import pallas as pl
from jax.experimental.pallas import tpu as pltpu
from jax.experimental.pallas import tpu_sc as plsc

N = 10000
E = 320000
NODE_DIM = 128
PE_DIM = 10
BOND_DIM = 16
HID = 128
N_BINS = 32
N_SITES = 10
PKA_MEAN = 7.0
PKA_STD = 3.0

_ROWS = 400
_RSQRT_H = float(1.0 / np.sqrt(float(HID)))

NW = 32
WIN = 64
DEPTH = 4
NPASS = 6
NCUT = NW * NPASS
STG = 16
RANGE_P = 64
E_PAD = E + WIN
_NCHUNK = HID // 16


def _t_cut(j):
    return 16 * ((j * (N // 16)) // NCUT)


def _h0_body(x_ref, lap_ref, wx_ref, wpe_ref, o_ref):
    o_ref[...] = (
        jnp.dot(x_ref[...], wx_ref[...], preferred_element_type=jnp.float32)
        + jnp.dot(lap_ref[...], wpe_ref[...], preferred_element_type=jnp.float32)
    )


def _h0_matmul(x, lap_pad, wx, wpe_pad):
    return pl.pallas_call(
        _h0_body,
        grid=(N // _ROWS,),
        in_specs=[
            pl.BlockSpec((_ROWS, NODE_DIM), lambda i: (i, 0)),
            pl.BlockSpec((_ROWS, 16), lambda i: (i, 0)),
            pl.BlockSpec((NODE_DIM, HID), lambda i: (0, 0)),
            pl.BlockSpec((16, HID), lambda i: (0, 0)),
        ],
        out_specs=pl.BlockSpec((_ROWS, HID), lambda i: (i, 0)),
        out_shape=jax.ShapeDtypeStruct((N, HID), jnp.float32),
    )(x, lap_pad, wx, wpe_pad)


def _qkvsu_body(h_ref, wq_ref, bq_ref, wkv_ref, bkv_ref, ws_ref, bs_ref,
                wet_ref, qu_ref, kv_ref, s_ref):
    h = h_ref[...]
    q = jnp.dot(h, wq_ref[...], preferred_element_type=jnp.float32) + bq_ref[...]
    u = jnp.dot(q, wet_ref[...], preferred_element_type=jnp.float32)
    qu_ref[...] = jnp.concatenate(
        [q, u, jnp.zeros((q.shape[0], 2 * HID - HID - BOND_DIM), jnp.float32)], axis=1)
    kv_ref[...] = jnp.dot(h, wkv_ref[...], preferred_element_type=jnp.float32) + bkv_ref[...]
    s_ref[...] = jnp.dot(h, ws_ref[...], preferred_element_type=jnp.float32) + bs_ref[...]


def _qkvsu_matmul(h, wq, bq, wkv, bkv, ws, bs, wet):
    return pl.pallas_call(
        _qkvsu_body,
        grid=(N // _ROWS,),
        in_specs=[
            pl.BlockSpec((_ROWS, HID), lambda i: (i, 0)),
            pl.BlockSpec((HID, HID), lambda i: (0, 0)),
            pl.BlockSpec((1, HID), lambda i: (0, 0)),
            pl.BlockSpec((HID, 2 * HID), lambda i: (0, 0)),
            pl.BlockSpec((1, 2 * HID), lambda i: (0, 0)),
            pl.BlockSpec((HID, HID), lambda i: (0, 0)),
            pl.BlockSpec((1, HID), lambda i: (0, 0)),
            pl.BlockSpec((HID, BOND_DIM), lambda i: (0, 0)),
        ],
        out_specs=[
            pl.BlockSpec((_ROWS, 2 * HID), lambda i: (i, 0)),
            pl.BlockSpec((_ROWS, 2 * HID), lambda i: (i, 0)),
            pl.BlockSpec((_ROWS, HID), lambda i: (i, 0)),
        ],
        out_shape=[
            jax.ShapeDtypeStruct((N, 2 * HID), jnp.float32),
            jax.ShapeDtypeStruct((N, 2 * HID), jnp.float32),
            jax.ShapeDtypeStruct((N, HID), jnp.float32),
        ],
    )(h, wq, bq, wkv, bkv, ws, bs, wet)


def _heads_body(h_ref, wc1_ref, bc1_ref, wc2_ref, bc2_ref,
                wr1_ref, br1_ref, wr2_ref, br2_ref, wg_ref, bg_ref,
                lg_ref, pr_ref, gt_ref):
    h = h_ref[...]
    z = jax.nn.relu(jnp.dot(h, wc1_ref[...], preferred_element_type=jnp.float32) + bc1_ref[...])
    lg_ref[...] = jnp.dot(z, wc2_ref[...], preferred_element_type=jnp.float32) + bc2_ref[...]
    r = jax.nn.relu(jnp.dot(h, wr1_ref[...], preferred_element_type=jnp.float32) + br1_ref[...])
    pr_ref[...] = jnp.dot(r, wr2_ref[...], preferred_element_type=jnp.float32) + br2_ref[...]
    gt_ref[...] = jnp.tanh(jnp.dot(h, wg_ref[...], preferred_element_type=jnp.float32) + bg_ref[...])


def _heads_matmul(h, wc1, bc1, wc2p, bc2p, wr1, br1, wr2p, br2p, wg, bg):
    return pl.pallas_call(
        _heads_body,
        grid=(N // _ROWS,),
        in_specs=[
            pl.BlockSpec((_ROWS, HID), lambda i: (i, 0)),
            pl.BlockSpec((HID, 128), lambda i: (0, 0)),
            pl.BlockSpec((1, 128), lambda i: (0, 0)),
            pl.BlockSpec((128, 128), lambda i: (0, 0)),
            pl.BlockSpec((1, 128), lambda i: (0, 0)),
            pl.BlockSpec((HID, 128), lambda i: (0, 0)),
            pl.BlockSpec((1, 128), lambda i: (0, 0)),
            pl.BlockSpec((128, 128), lambda i: (0, 0)),
            pl.BlockSpec((1, 128), lambda i: (0, 0)),
            pl.BlockSpec((HID, HID), lambda i: (0, 0)),
            pl.BlockSpec((1, HID), lambda i: (0, 0)),
        ],
        out_specs=[
            pl.BlockSpec((_ROWS, 128), lambda i: (i, 0)),
            pl.BlockSpec((_ROWS, 128), lambda i: (i, 0)),
            pl.BlockSpec((_ROWS, HID), lambda i: (i, 0)),
        ],
        out_shape=[
            jax.ShapeDtypeStruct((N, 128), jnp.float32),
            jax.ShapeDtypeStruct((N, 128), jnp.float32),
            jax.ShapeDtypeStruct((N, HID), jnp.float32),
        ],
    )(h, wc1, bc1, wc2p, bc2p, wr1, br1, wr2p, br2p, wg, bg)


def _combine_body(s1_ref, s2_ref, we_ref, skip_ref, o_ref):
    o_ref[...] = (
        s1_ref[...]
        + jnp.dot(s2_ref[...][:, :BOND_DIM], we_ref[...],
                  preferred_element_type=jnp.float32)
        + skip_ref[...]
    )


def _combine_matmul(s1, s2, we, skip):
    return pl.pallas_call(
        _combine_body,
        grid=(N // _ROWS,),
        in_specs=[
            pl.BlockSpec((_ROWS, HID), lambda i: (i, 0)),
            pl.BlockSpec((_ROWS, 128), lambda i: (i, 0)),
            pl.BlockSpec((BOND_DIM, HID), lambda i: (0, 0)),
            pl.BlockSpec((_ROWS, HID), lambda i: (i, 0)),
        ],
        out_specs=pl.BlockSpec((_ROWS, HID), lambda i: (i, 0)),
        out_shape=jax.ShapeDtypeStruct((N, HID), jnp.float32),
    )(s1, s2, we, skip)


_MESH = plsc.VectorSubcoreMesh(core_axis_name="c", subcore_axis_name="s")


@functools.partial(
    pl.kernel,
    out_type=[
        jax.ShapeDtypeStruct((N, HID), jnp.float32),
        jax.ShapeDtypeStruct((N, 128), jnp.float32),
    ],
    mesh=_MESH,
    scratch_types=[
        pltpu.VMEM((RANGE_P, 2 * HID), jnp.float32),
        pltpu.VMEM((RANGE_P, HID), jnp.float32),
        pltpu.VMEM((RANGE_P, 128), jnp.float32),
        pltpu.VMEM((DEPTH, WIN), jnp.int32),
        pltpu.VMEM((DEPTH, WIN), jnp.int32),
        pltpu.VMEM((DEPTH * WIN * BOND_DIM,), jnp.float32),
        pltpu.VMEM((DEPTH, WIN, 2 * HID), jnp.float32),
        pltpu.VMEM((224,), jnp.int32),
        pltpu.SemaphoreType.DMA,
        pltpu.SemaphoreType.DMA,
        pltpu.SemaphoreType.DMA,
        pltpu.SemaphoreType.DMA,
        pltpu.SemaphoreType.DMA,
        pltpu.SemaphoreType.DMA,
        pltpu.SemaphoreType.DMA,
        pltpu.SemaphoreType.DMA,
    ],
)
def _edge_sc(qu_hbm, src_hbm, dst_hbm, ea_hbm, kv_hbm, starts_hbm,
             s1_hbm, s2_hbm,
             qub, t1b, auxb, srcw, dstw, eaw, kvw, stbl,
             sem_src0, sem_src1, sem_src2, sem_src3,
             sem_kv0, sem_kv1, sem_kv2, sem_kv3):
    wid = lax.axis_index("s") * 2 + lax.axis_index("c")

    pltpu.sync_copy(starts_hbm, stbl)

    sem_src = (sem_src0, sem_src1, sem_src2, sem_src3)
    sem_kv = (sem_kv0, sem_kv1, sem_kv2, sem_kv3)
    zrow = jnp.zeros((16,), jnp.float32)
    mrow = jnp.full((16,), -3e38, jnp.float32)

    def run_pass(pi, carry):
        j = wid * NPASS + pi
        t_lo = 16 * ((j * (N // 16)) // NCUT)
        t_hi = 16 * (((j + 1) * (N // 16)) // NCUT)
        sv = stbl[pl.ds(j, 16)]
        a_w = (sv[0] // 8) * 8
        end_w = sv[1]
        nw = (end_w - a_w + (WIN - 1)) // WIN

        pltpu.sync_copy(qu_hbm.at[pl.ds(t_lo, RANGE_P)], qub)

        def init_body(r, c2):
            for c in range(_NCHUNK):
                t1b[r, pl.ds(16 * c, 16)] = zrow
            auxb[r, pl.ds(0, 16)] = zrow
            auxb[r, pl.ds(16, 16)] = mrow
            auxb[r, pl.ds(32, 16)] = zrow
            return c2

        lax.fori_loop(0, RANGE_P, init_body, 0)

        def issue_src(g, par):
            e0 = a_w + g * WIN
            pltpu.make_async_copy(src_hbm.at[pl.ds(e0, WIN)], srcw.at[par],
                                  sem_src[par]).start()
            pltpu.make_async_copy(dst_hbm.at[pl.ds(e0, WIN)], dstw.at[par],
                                  sem_src[par]).start()
            pltpu.make_async_copy(ea_hbm.at[pl.ds(e0 * BOND_DIM, WIN * BOND_DIM)],
                                  eaw.at[pl.ds(par * WIN * BOND_DIM, WIN * BOND_DIM)],
                                  sem_src[par]).start()

        def wait_src(par):
            pltpu.make_async_copy(src_hbm.at[pl.ds(0, WIN)], srcw.at[par],
                                  sem_src[par]).wait()
            pltpu.make_async_copy(dst_hbm.at[pl.ds(0, WIN)], dstw.at[par],
                                  sem_src[par]).wait()
            pltpu.make_async_copy(ea_hbm.at[pl.ds(0, WIN * BOND_DIM)],
                                  eaw.at[pl.ds(par * WIN * BOND_DIM, WIN * BOND_DIM)],
                                  sem_src[par]).wait()

        def issue_kv(par):
            pltpu.make_async_copy(kv_hbm.at[srcw.at[par]], kvw.at[par],
                                  sem_kv[par]).start()

        def wait_kv(par):
            pltpu.make_async_copy(kv_hbm.at[srcw.at[par]], kvw.at[par],
                                  sem_kv[par]).wait()

        def win_body(g, par):
            wait_kv(par)

            def group_body(gi, c2):
                base = gi * 16
                dvec = dstw[par, pl.ds(base, 16)]
                lane = lax.iota(jnp.int32, 16)

                alpha16 = zrow
                d_eff = []
                for l in range(16):
                    i = base + l
                    d = dvec[l]
                    valid = jnp.logical_and(d >= t_lo, d < t_hi)
                    row = jnp.where(valid, d - t_lo, 0)
                    d_eff.append(jnp.minimum(jnp.maximum(d, t_lo), t_hi - 1))
                    ea_row = eaw[pl.ds(par * WIN * BOND_DIM + i * BOND_DIM, 16)]
                    acc = qub[row, pl.ds(HID, BOND_DIM)] * ea_row
                    for c in range(_NCHUNK):
                        acc = acc + qub[row, pl.ds(16 * c, 16)] * kvw[par, i, pl.ds(16 * c, 16)]
                    for k in (8, 4, 2, 1):
                        acc = acc + acc.at[lane ^ k].get(mode="promise_in_bounds")
                    vf = jnp.full((16,), jnp.where(valid, 1.0, 0.0), jnp.float32)
                    a_spl = acc * _RSQRT_H * vf + mrow * (1.0 - vf)
                    oh = jnp.maximum(1.0 - jnp.abs(lane - l).astype(jnp.float32), 0.0)
                    alpha16 = alpha16 + a_spl * oh

                mg = alpha16
                for k in (8, 4, 2, 1):
                    mg = jnp.maximum(mg, mg.at[lane ^ k].get(mode="promise_in_bounds"))
                gate = jnp.full(
                    (16,), jnp.where(mg[0] > -1e30, 1.0, 0.0), jnp.float32)
                exg = jnp.exp(alpha16 - mg) * gate

                def flush(prow, st):
                    accD, accT2 = st[0], st[1]
                    accT1 = st[2:]
                    row = prow - t_lo
                    m_old = auxb[row, pl.ds(16, 16)]
                    m_new = jnp.maximum(m_old, mg)
                    f_old = jnp.exp(m_old - m_new)
                    f_g = jnp.exp(mg - m_new)
                    auxb[row, pl.ds(16, 16)] = m_new
                    auxb[row, pl.ds(32, 16)] = auxb[row, pl.ds(32, 16)] * f_old + accD * f_g
                    auxb[row, pl.ds(0, 16)] = auxb[row, pl.ds(0, 16)] * f_old + accT2 * f_g
                    for c in range(_NCHUNK):
                        t1b[row, pl.ds(16 * c, 16)] = (
                            t1b[row, pl.ds(16 * c, 16)] * f_old + accT1[c] * f_g
                        )
                    return (zrow,) * (2 + _NCHUNK)

                st = (zrow,) * (2 + _NCHUNK)
                for l in range(16):
                    i = base + l
                    if l > 0:
                        st = lax.cond(d_eff[l] != d_eff[l - 1],
                                      lambda s, _r=d_eff[l - 1]: flush(_r, s),
                                      lambda s: s, st)
                    ex_s = jnp.full((16,), exg[l], jnp.float32)
                    ea_row = eaw[pl.ds(par * WIN * BOND_DIM + i * BOND_DIM, 16)]
                    new0 = st[0] + ex_s
                    new1 = st[1] + ex_s * ea_row
                    newT = tuple(
                        st[2 + c] + ex_s * kvw[par, i, pl.ds(HID + 16 * c, 16)]
                        for c in range(_NCHUNK))
                    st = (new0, new1) + newT
                flush(d_eff[15], st)
                return c2

            lax.fori_loop(0, WIN // 16, group_body, 0)

            @pl.when(g + DEPTH < nw)
            def _():
                issue_src(g + DEPTH, par)

            @pl.when(g + 3 < nw)
            def _():
                wait_src((par + 3) % DEPTH)
                issue_kv((par + 3) % DEPTH)

        for i0 in range(DEPTH):
            @pl.when(i0 < nw)
            def _():
                issue_src(i0, i0)

        for i0 in range(3):
            @pl.when(i0 < nw)
            def _():
                wait_src(i0)
                issue_kv(i0)

        def quad_body(gq, c2):
            g = DEPTH * gq
            for p in range(DEPTH):
                @pl.when(g + p < nw)
                def _():
                    win_body(g + p, p)
            return c2

        lax.fori_loop(0, (nw + DEPTH - 1) // DEPTH, quad_body, 0)

        def div_body(r, c2):
            inv = 1.0 / (auxb[r, pl.ds(32, 16)] + 1e-16)
            for c in range(_NCHUNK):
                t1b[r, pl.ds(16 * c, 16)] = t1b[r, pl.ds(16 * c, 16)] * inv
            auxb[r, pl.ds(0, 16)] = auxb[r, pl.ds(0, 16)] * inv
            return c2

        lax.fori_loop(0, t_hi - t_lo, div_body, 0)

        def flush_body(fb, c2):
            rb = fb * STG
            pltpu.sync_copy(t1b.at[pl.ds(rb, STG)], s1_hbm.at[pl.ds(t_lo + rb, STG)])
            pltpu.sync_copy(auxb.at[pl.ds(rb, STG)], s2_hbm.at[pl.ds(t_lo + rb, STG)])
            return c2

        lax.fori_loop(0, (t_hi - t_lo) // STG, flush_body, 0)
        return carry

    lax.fori_loop(0, NPASS, run_pass, 0)


def _prep(pka_labels):
    pos = pka_labels > 0
    sort_key = jnp.where(pos, pka_labels, jnp.inf)
    idx_sorted = jnp.argsort(sort_key, stable=True)[:N_SITES].astype(jnp.int32)
    target0 = pos.astype(jnp.int32)

    def step(cur, i):
        nxt = cur.at[i].set(0)
        return nxt, nxt

    _, rows = jax.lax.scan(step, target0, idx_sorted)
    target_final = jnp.concatenate([target0[None], rows], axis=0)
    return idx_sorted, target_final


def kernel(x, lap_pos, edge_attr, params, edge_index, pka_labels):
    p = params
    idx_sorted, target_final = _prep(pka_labels)
    steps = N_SITES + 1

    wkv = jnp.concatenate([p['Wk'], p['Wv']], axis=1)
    bkv = jnp.concatenate([p['bk'], p['bv']])[None, :]
    wc2p = jnp.zeros((128, 128), jnp.float32).at[:, :2].set(p['Wc2'])
    bc2p = jnp.zeros((1, 128), jnp.float32).at[0, :2].set(p['bc2'])
    wr2p = jnp.zeros((128, 128), jnp.float32).at[:, :1].set(p['Wr2'])
    br2p = jnp.zeros((1, 128), jnp.float32).at[0, :1].set(p['br2'])
    lap_pad = jnp.zeros((N, 16), jnp.float32).at[:, :PE_DIM].set(lap_pos)
    wpe_pad = jnp.zeros((16, HID), jnp.float32).at[:PE_DIM, :].set(p['W_pe'])
    wet = p['We'].T

    src0 = edge_index[0]
    dst0 = edge_index[1]
    order = jnp.argsort(dst0, stable=True)
    src = jnp.concatenate([src0[order], jnp.zeros((WIN,), jnp.int32)])
    dst = jnp.concatenate([dst0[order], jnp.full((WIN,), N, jnp.int32)])
    ea_s = jnp.concatenate(
        [edge_attr[order], jnp.zeros((WIN, BOND_DIM), jnp.float32)]).reshape(-1)
    t_cuts = jnp.asarray([_t_cut(j) for j in range(NCUT + 1)], jnp.int32)
    starts = jnp.searchsorted(dst[:E], t_cuts).astype(jnp.int32)
    starts = jnp.concatenate([starts, jnp.zeros((224 - (NCUT + 1),), jnp.int32)])

    h0 = _h0_matmul(x, lap_pad, p['W_x'], wpe_pad)

    def conv(h_in):
        qu, kv, skip = _qkvsu_matmul(
            h_in, p['Wq'], p['bq'][None, :], wkv, bkv,
            p['Wskip'], p['bskip'][None, :], wet)
        s1, s2 = _edge_sc(qu, src, dst, ea_s, kv, starts)
        return _combine_matmul(s1, s2, p['We'], skip)

    h = conv(h0)

    loss_cla_steps = []
    loss_reg_steps = []
    logitss = []
    pkas = []
    for si in range(steps):
        idx = idx_sorted[si] if si != steps - 1 else jnp.int32(-1)
        lg_p, pr_p, gate_all = _heads_matmul(
            h, p['Wc1'], p['bc1'][None, :], wc2p, bc2p,
            p['Wr1'], p['br1'][None, :], wr2p, br2p, p['Wg'], p['bg'][None, :])
        logits = lg_p[:, :2]
        logitss.append(logits)
        tgt = target_final[si]
        n1 = jnp.sum(tgt).astype(jnp.float32)
        n0 = jnp.sum(tgt == 0).astype(jnp.float32)
        ratio = n0 / (n1 + 1e-06)
        logp = jax.nn.log_softmax(logits, axis=-1)
        nll = -jnp.take_along_axis(logp, tgt[:, None], axis=1)[:, 0]
        wpn = jnp.where(tgt == 1, ratio, 1.0)
        loss_cla_steps.append(wpn * nll)
        if si != steps - 1:
            pred_pka = pr_p[idx, 0]
            true_pka = pka_labels[idx]
            bi = jnp.searchsorted(jax.lax.stop_gradient(p['bin_edges']),
                                  jax.lax.stop_gradient(true_pka), side='left') - 1
            w = p['bin_weights'][bi]
            loss_r = w * (((pred_pka - PKA_MEAN) / PKA_STD - (true_pka - PKA_MEAN) / PKA_STD) ** 2)
            loss_reg_steps.append(loss_r)
            pkas.append(pred_pka)
        gate = gate_all[idx]
        h_upd = h.at[idx].set(h0[idx] + h[idx] * gate)
        if si != steps - 1:
            h = conv(h_upd)
    loss_cla = jnp.mean(jnp.stack(loss_cla_steps))
    loss_reg = jnp.mean(jnp.stack(loss_reg_steps))
    total = loss_cla + loss_reg
    return total, loss_cla, loss_reg, jnp.stack(logitss), jnp.stack(pkas), target_final
